# baseline (speedup 1.0000x reference)
import jax
import jax.numpy as jnp
from jax import lax
from jax.experimental import pallas as pl
from jax.experimental.pallas import tpu as pltpu

NUM_EXPERTS = 8
HIDDEN = 768
CHUNK = 2048
NBUF = 4


def _router_body(x_hbm, wt_ref, p1_ref, p2_ref, oh_ref, bufs, sems):
    r = pl.program_id(0)
    nc = pl.num_programs(0) * NBUF

    def start(c, k):
        pltpu.make_async_copy(
            x_hbm.at[pl.ds(c * CHUNK, CHUNK), :],
            bufs.at[k],
            sems.at[k],
        ).start()

    @pl.when(r == 0)
    def _prime():
        for k in range(NBUF):
            start(k, k)

    for k in range(NBUF):
        c = r * NBUF + k
        pltpu.make_async_copy(
            x_hbm.at[pl.ds(c * CHUNK, CHUNK), :], bufs.at[k], sems.at[k]
        ).wait()
        logits = lax.dot_general(
            wt_ref[...],
            bufs[k],
            (((1,), (1,)), ((), ())),
            preferred_element_type=jnp.float32,
        )
        m = jnp.max(logits, axis=0, keepdims=True)
        unn = jnp.exp(logits - m)
        s = jnp.sum(unn, axis=0, keepdims=True)
        probs = unn / s
        maxp = jnp.max(probs, axis=0, keepdims=True)
        sub = lax.broadcasted_iota(jnp.int32, probs.shape, 0)
        idx = jnp.min(jnp.where(probs == maxp, sub, NUM_EXPERTS), axis=0,
                      keepdims=True)
        maxp_pk = maxp.reshape(CHUNK // 128, 128)
        p1_ref[pl.ds(k * (CHUNK // 128), CHUNK // 128), :] = maxp_pk
        p2_ref[pl.ds(k * (CHUNK // 128), CHUNK // 128), :] = maxp_pk
        oh_ref[:, pl.ds(k * CHUNK, CHUNK)] = (sub == idx).astype(jnp.int32)

        @pl.when(c + NBUF < nc)
        def _next():
            start(c + NBUF, k)


def kernel(hidden_states, W):
    B, S, H = hidden_states.shape
    T = B * S
    x = hidden_states.reshape(T, H)
    grid = (T // (CHUNK * NBUF),)
    p1, p2, oh = pl.pallas_call(
        _router_body,
        grid=grid,
        in_specs=[
            pl.BlockSpec(memory_space=pltpu.MemorySpace.HBM),
            pl.BlockSpec((NUM_EXPERTS, HIDDEN), lambda i: (0, 0)),
        ],
        out_specs=[
            pl.BlockSpec((CHUNK * NBUF // 128, 128), lambda i: (i, 0)),
            pl.BlockSpec((CHUNK * NBUF // 128, 128), lambda i: (i, 0)),
            pl.BlockSpec((NUM_EXPERTS, CHUNK * NBUF), lambda i: (0, i)),
        ],
        out_shape=[
            jax.ShapeDtypeStruct((T // 128, 128), jnp.float32),
            jax.ShapeDtypeStruct((T // 128, 128), jnp.float32),
            jax.ShapeDtypeStruct((NUM_EXPERTS, T), jnp.int32),
        ],
        scratch_shapes=[
            pltpu.VMEM((NBUF, CHUNK, HIDDEN), jnp.float32),
            pltpu.SemaphoreType.DMA((NBUF,)),
        ],
        compiler_params=pltpu.CompilerParams(
            dimension_semantics=("arbitrary",),
        ),
    )(x, W.T)
    p_out = p1.reshape(B, S, 1)
    p2_out = p2.reshape(B, S, 1)
    oh_out = oh.T.reshape(B, S, 1, NUM_EXPERTS).astype(jnp.int64)
    return (p_out, oh_out, p2_out)


# speedup vs baseline: 1.1512x; 1.1512x over previous
import jax
import jax.numpy as jnp
from jax import lax
from jax.experimental import pallas as pl
from jax.experimental.pallas import tpu as pltpu

NUM_EXPERTS = 8
HIDDEN = 768
CHUNK = 512
NBUF = 6


def _router_body(x_hbm, wt_ref, p1_ref, p2_ref, oh_ref, bufs, sems):
    r = pl.program_id(0)
    nc = pl.num_programs(0) * NBUF

    def start(c, k):
        pltpu.make_async_copy(
            x_hbm.at[pl.ds(c * CHUNK, CHUNK), :],
            bufs.at[k],
            sems.at[k],
        ).start()

    @pl.when(r == 0)
    def _prime():
        for k in range(NBUF):
            start(k, k)

    for k in range(NBUF):
        c = r * NBUF + k
        pltpu.make_async_copy(
            x_hbm.at[pl.ds(c * CHUNK, CHUNK), :], bufs.at[k], sems.at[k]
        ).wait()
        logits = lax.dot_general(
            wt_ref[...],
            bufs[k],
            (((1,), (1,)), ((), ())),
            preferred_element_type=jnp.float32,
        )
        m = jnp.max(logits, axis=0, keepdims=True)
        unn = jnp.exp(logits - m)
        s = jnp.sum(unn, axis=0, keepdims=True)
        probs = unn / s
        maxp = jnp.max(probs, axis=0, keepdims=True)
        sub = lax.broadcasted_iota(jnp.int32, probs.shape, 0)
        idx = jnp.min(jnp.where(probs == maxp, sub, NUM_EXPERTS), axis=0,
                      keepdims=True)
        maxp_pk = maxp.reshape(CHUNK // 128, 128)
        p1_ref[pl.ds(k * (CHUNK // 128), CHUNK // 128), :] = maxp_pk
        p2_ref[pl.ds(k * (CHUNK // 128), CHUNK // 128), :] = maxp_pk
        oh_ref[:, pl.ds(k * CHUNK, CHUNK)] = (sub == idx).astype(jnp.int32)

        @pl.when(c + NBUF < nc)
        def _next():
            start(c + NBUF, k)


def kernel(hidden_states, W):
    B, S, H = hidden_states.shape
    T = B * S
    x = hidden_states.reshape(T, H)
    grid = (T // (CHUNK * NBUF),)
    p1, p2, oh = pl.pallas_call(
        _router_body,
        grid=grid,
        in_specs=[
            pl.BlockSpec(memory_space=pltpu.MemorySpace.HBM),
            pl.BlockSpec((NUM_EXPERTS, HIDDEN), lambda i: (0, 0)),
        ],
        out_specs=[
            pl.BlockSpec((CHUNK * NBUF // 128, 128), lambda i: (i, 0)),
            pl.BlockSpec((CHUNK * NBUF // 128, 128), lambda i: (i, 0)),
            pl.BlockSpec((NUM_EXPERTS, CHUNK * NBUF), lambda i: (0, i)),
        ],
        out_shape=[
            jax.ShapeDtypeStruct((T // 128, 128), jnp.float32),
            jax.ShapeDtypeStruct((T // 128, 128), jnp.float32),
            jax.ShapeDtypeStruct((NUM_EXPERTS, T), jnp.int32),
        ],
        scratch_shapes=[
            pltpu.VMEM((NBUF, CHUNK, HIDDEN), jnp.float32),
            pltpu.SemaphoreType.DMA((NBUF,)),
        ],
        compiler_params=pltpu.CompilerParams(
            dimension_semantics=("arbitrary",),
        ),
    )(x, W.T)
    p_out = p1.reshape(B, S, 1)
    p2_out = p2.reshape(B, S, 1)
    oh_out = oh.T.reshape(B, S, 1, NUM_EXPERTS).astype(jnp.int64)
    return (p_out, oh_out, p2_out)
